# baseline (speedup 1.0000x reference)
import functools

import jax
import jax.numpy as jnp
from jax import lax
from jax.experimental import pallas as pl
from jax.experimental.pallas import tpu as pltpu
from jax.experimental.pallas import tpu_sc as plsc

_NW = 32
_T = 64


def _wid():
    return lax.axis_index("s") * 2 + lax.axis_index("c")


def _build_inv(idx_hbm, idx_v, inv_v, nk):
    pltpu.sync_copy(idx_hbm, idx_v)
    iota = lax.iota(jnp.int32, 16)

    def body(k, carry):
        for c in range(8):
            q = idx_v[k, pl.ds(c * 16, 16)]
            plsc.store_scatter(inv_v, [q], iota + (k * 128 + c * 16))
        return carry

    lax.fori_loop(0, nk, body, 0)


def _sc_scatter_x(x, idx2, nslots):
    total, d = x.shape
    nk = idx2.shape[0]
    rpw = total // _NW
    nq = rpw // 16
    mesh = plsc.VectorSubcoreMesh(core_axis_name="c", subcore_axis_name="s")

    @functools.partial(
        pl.kernel,
        mesh=mesh,
        out_type=jax.ShapeDtypeStruct((nslots, d), jnp.float32),
        scratch_types=[
            pltpu.VMEM((nk, 128), jnp.int32),
            pltpu.VMEM((total + 16,), jnp.int32),
            pltpu.VMEM((rpw, d), jnp.float32),
            pltpu.SemaphoreType.DMA,
            pltpu.SemaphoreType.DMA,
        ],
        compiler_params=pltpu.CompilerParams(needs_layout_passes=False),
    )
    def k(x_hbm, idx_hbm, out_hbm, idx_v, inv_v, buf, rsem, wsem):
        w = _wid()
        rd = pltpu.async_copy(x_hbm.at[pl.ds(w * rpw, rpw)], buf, rsem)
        _build_inv(idx_hbm, idx_v, inv_v, nk)
        rd.wait()
        iota = lax.iota(jnp.int32, 16)
        hs = []
        for c in range(nq):
            q = plsc.load_gather(inv_v, [iota + (w * rpw + c * 16)])
            hs.append(pltpu.async_copy(
                buf.at[pl.ds(c * 16, 16)], out_hbm.at[q], wsem))
        for h in hs:
            h.wait()

    return k(x, idx2)


def _sc_gather_out(path_flat, idx2, total):
    d = path_flat.shape[1]
    nk = idx2.shape[0]
    rpw = total // _NW
    nq = rpw // 16
    mesh = plsc.VectorSubcoreMesh(core_axis_name="c", subcore_axis_name="s")

    @functools.partial(
        pl.kernel,
        mesh=mesh,
        out_type=jax.ShapeDtypeStruct((total, d), jnp.float32),
        scratch_types=[
            pltpu.VMEM((nk, 128), jnp.int32),
            pltpu.VMEM((total + 16,), jnp.int32),
            pltpu.VMEM((rpw, d), jnp.float32),
            pltpu.SemaphoreType.DMA,
        ],
        compiler_params=pltpu.CompilerParams(needs_layout_passes=False),
    )
    def k(path_hbm, idx_hbm, out_hbm, idx_v, inv_v, buf, sem):
        w = _wid()
        _build_inv(idx_hbm, idx_v, inv_v, nk)
        iota = lax.iota(jnp.int32, 16)
        hs = []
        for c in range(nq):
            q = plsc.load_gather(inv_v, [iota + (w * rpw + c * 16)])
            hs.append(pltpu.async_copy(
                path_hbm.at[q], buf.at[pl.ds(c * 16, 16)], sem))
        for h in hs:
            h.wait()
        pltpu.sync_copy(buf, out_hbm.at[pl.ds(w * rpw, rpw)])

    return k(path_flat, idx2)


def _scan_body(u_ref, wi_ref, bi_ref, wh_ref, bh_ref, len_ref, o_ref,
               xw_ref, h_ref):
    pid = pl.program_id(0)

    @pl.when(pid == 0)
    def _():
        h_ref[...] = jnp.zeros_like(h_ref)

    b = o_ref.shape[1]
    d = wh_ref.shape[0]
    u = u_ref[...].reshape(_T * b, d).astype(jnp.bfloat16)
    xw_ref[...] = (
        jnp.dot(u, wi_ref[...], preferred_element_type=jnp.float32)
        + bi_ref[...]
    )
    wh = wh_ref[...]
    bh = bh_ref[...]
    lens = len_ref[...]

    def step(t, h):
        xw = xw_ref[pl.ds(t * b, b), :]
        hU = jnp.dot(h.astype(jnp.bfloat16), wh,
                     preferred_element_type=jnp.float32) + bh
        r = jax.nn.sigmoid(xw[:, :d] + hU[:, :d])
        z = jax.nn.sigmoid(xw[:, d:2 * d] + hU[:, d:2 * d])
        n = jnp.tanh(xw[:, 2 * d:] + r * hU[:, 2 * d:])
        hnew = (1.0 - z) * n + z * h
        m = lens > (pid * _T + t)
        o_ref[t] = jnp.where(m, hnew, 0.0)
        return jnp.where(m, hnew, h)

    h_ref[...] = lax.fori_loop(0, _T, step, h_ref[...])


def _gru_scan(unflat, W_i, b_i, W_h, b_h, lengths):
    l, b, d = unflat.shape
    n3 = W_i.shape[1]
    lens = jnp.broadcast_to(lengths.astype(jnp.int32)[:, None], (b, d))
    return pl.pallas_call(
        _scan_body,
        grid=(l // _T,),
        in_specs=[
            pl.BlockSpec((_T, b, d), lambda i: (i, 0, 0)),
            pl.BlockSpec((d, n3), lambda i: (0, 0)),
            pl.BlockSpec((1, n3), lambda i: (0, 0)),
            pl.BlockSpec((d, n3), lambda i: (0, 0)),
            pl.BlockSpec((1, n3), lambda i: (0, 0)),
            pl.BlockSpec((b, d), lambda i: (0, 0)),
        ],
        out_specs=pl.BlockSpec((_T, b, d), lambda i: (i, 0, 0)),
        out_shape=jax.ShapeDtypeStruct((l, b, d), jnp.float32),
        scratch_shapes=[
            pltpu.VMEM((_T * b, n3), jnp.float32),
            pltpu.VMEM((b, d), jnp.float32),
        ],
        compiler_params=pltpu.CompilerParams(
            dimension_semantics=("arbitrary",)),
    )(unflat, W_i.astype(jnp.bfloat16), b_i.reshape(1, n3),
      W_h.astype(jnp.bfloat16), b_h.reshape(1, n3), lens)


def kernel(cfg_nodes_encodings, W_i, W_h, b_i, b_h, permutations,
           unflattener_mask, lengths):
    x = cfg_nodes_encodings
    total, d = x.shape
    bsz, l = permutations.shape

    perm = permutations.astype(jnp.int32)
    idx2 = jnp.where(unflattener_mask, perm, total).T.reshape(
        (l * bsz) // 128, 128).astype(jnp.int32)

    unflat = _sc_scatter_x(x, idx2, l * bsz)
    path = _gru_scan(unflat.reshape(l, bsz, d), W_i, b_i, W_h, b_h, lengths)
    return _sc_gather_out(path.reshape(l * bsz, d), idx2, total)


# speedup vs baseline: 1.0377x; 1.0377x over previous
import functools

import jax
import jax.numpy as jnp
from jax import lax
from jax.experimental import pallas as pl
from jax.experimental.pallas import tpu as pltpu
from jax.experimental.pallas import tpu_sc as plsc

_NW = 32
_T = 64


def _wid():
    return lax.axis_index("s") * 2 + lax.axis_index("c")


def _build_inv(idx_hbm, idx_v, inv_v, nk):
    pltpu.sync_copy(idx_hbm, idx_v)
    iota = lax.iota(jnp.int32, 16)

    def body(k, carry):
        for c in range(8):
            q = idx_v[k, pl.ds(c * 16, 16)]
            plsc.store_scatter(inv_v, [q], iota + (k * 128 + c * 16))
        return carry

    lax.fori_loop(0, nk, body, 0)


def _sc_scatter_x(x, idx2, nslots):
    total, d = x.shape
    nk = idx2.shape[0]
    rpw = total // _NW
    nq = rpw // 16
    mesh = plsc.VectorSubcoreMesh(core_axis_name="c", subcore_axis_name="s")

    @functools.partial(
        pl.kernel,
        mesh=mesh,
        out_type=jax.ShapeDtypeStruct((nslots, d), jnp.float32),
        scratch_types=[
            pltpu.VMEM((nk, 128), jnp.int32),
            pltpu.VMEM((total + 16,), jnp.int32),
            pltpu.VMEM((rpw, d), jnp.float32),
            pltpu.SemaphoreType.DMA,
            pltpu.SemaphoreType.DMA,
        ],
        compiler_params=pltpu.CompilerParams(needs_layout_passes=False),
    )
    def k(x_hbm, idx_hbm, out_hbm, idx_v, inv_v, buf, rsem, wsem):
        w = _wid()
        rd = pltpu.async_copy(x_hbm.at[pl.ds(w * rpw, rpw)], buf, rsem)
        _build_inv(idx_hbm, idx_v, inv_v, nk)
        rd.wait()
        iota = lax.iota(jnp.int32, 16)
        hs = []
        for c in range(nq):
            q = plsc.load_gather(inv_v, [iota + (w * rpw + c * 16)])
            hs.append(pltpu.async_copy(
                buf.at[pl.ds(c * 16, 16)], out_hbm.at[q], wsem))
        for h in hs:
            h.wait()

    return k(x, idx2)


def _sc_gather_out(path_flat, idx2, total):
    d = path_flat.shape[1]
    nk = idx2.shape[0]
    rpw = total // _NW
    nq = rpw // 16
    mesh = plsc.VectorSubcoreMesh(core_axis_name="c", subcore_axis_name="s")

    @functools.partial(
        pl.kernel,
        mesh=mesh,
        out_type=jax.ShapeDtypeStruct((total, d), jnp.float32),
        scratch_types=[
            pltpu.VMEM((nk, 128), jnp.int32),
            pltpu.VMEM((total + 16,), jnp.int32),
            pltpu.VMEM((rpw, d), jnp.float32),
            pltpu.SemaphoreType.DMA,
        ],
        compiler_params=pltpu.CompilerParams(needs_layout_passes=False),
    )
    def k(path_hbm, idx_hbm, out_hbm, idx_v, inv_v, buf, sem):
        w = _wid()
        _build_inv(idx_hbm, idx_v, inv_v, nk)
        iota = lax.iota(jnp.int32, 16)
        hs = []
        for c in range(nq):
            q = plsc.load_gather(inv_v, [iota + (w * rpw + c * 16)])
            hs.append(pltpu.async_copy(
                path_hbm.at[q], buf.at[pl.ds(c * 16, 16)], sem))
        for h in hs:
            h.wait()
        pltpu.sync_copy(buf, out_hbm.at[pl.ds(w * rpw, rpw)])

    return k(path_flat, idx2)


def _scan_body(u_ref, wi_ref, bi_ref, wh_ref, bh_ref, o_ref, xw_ref, h_ref):
    pid = pl.program_id(0)

    @pl.when(pid == 0)
    def _():
        h_ref[...] = jnp.zeros_like(h_ref)

    b = o_ref.shape[1]
    d = wh_ref.shape[0]
    u = u_ref[...].reshape(_T * b, d).astype(jnp.bfloat16)
    xw_ref[...] = (
        jnp.dot(u, wi_ref[...], preferred_element_type=jnp.float32)
        + bi_ref[...]
    )
    wh = wh_ref[...]
    bh = bh_ref[...]

    def step(t, h):
        xw = xw_ref[pl.ds(t * b, b), :]
        hU = jnp.dot(h.astype(jnp.bfloat16), wh,
                     preferred_element_type=jnp.float32) + bh
        r = jax.nn.sigmoid(xw[:, :d] + hU[:, :d])
        z = jax.nn.sigmoid(xw[:, d:2 * d] + hU[:, d:2 * d])
        n = jnp.tanh(xw[:, 2 * d:] + r * hU[:, 2 * d:])
        hnew = (1.0 - z) * n + z * h
        o_ref[t] = hnew
        return hnew

    h_ref[...] = lax.fori_loop(0, _T, step, h_ref[...], unroll=2)


def _gru_scan(unflat, W_i, b_i, W_h, b_h):
    l, b, d = unflat.shape
    n3 = W_i.shape[1]
    return pl.pallas_call(
        _scan_body,
        grid=(l // _T,),
        in_specs=[
            pl.BlockSpec((_T, b, d), lambda i: (i, 0, 0)),
            pl.BlockSpec((d, n3), lambda i: (0, 0)),
            pl.BlockSpec((1, n3), lambda i: (0, 0)),
            pl.BlockSpec((d, n3), lambda i: (0, 0)),
            pl.BlockSpec((1, n3), lambda i: (0, 0)),
        ],
        out_specs=pl.BlockSpec((_T, b, d), lambda i: (i, 0, 0)),
        out_shape=jax.ShapeDtypeStruct((l, b, d), jnp.float32),
        scratch_shapes=[
            pltpu.VMEM((_T * b, n3), jnp.float32),
            pltpu.VMEM((b, d), jnp.float32),
        ],
        compiler_params=pltpu.CompilerParams(
            dimension_semantics=("arbitrary",)),
    )(unflat, W_i.astype(jnp.bfloat16), b_i.reshape(1, n3),
      W_h.astype(jnp.bfloat16), b_h.reshape(1, n3))


def kernel(cfg_nodes_encodings, W_i, W_h, b_i, b_h, permutations,
           unflattener_mask, lengths):
    x = cfg_nodes_encodings
    total, d = x.shape
    bsz, l = permutations.shape

    perm = permutations.astype(jnp.int32)
    idx2 = jnp.where(unflattener_mask, perm, total).T.reshape(
        (l * bsz) // 128, 128).astype(jnp.int32)

    unflat = _sc_scatter_x(x, idx2, l * bsz)
    path = _gru_scan(unflat.reshape(l, bsz, d), W_i, b_i, W_h, b_h)
    return _sc_gather_out(path.reshape(l * bsz, d), idx2, total)
